# Initial kernel scaffold; baseline (speedup 1.0000x reference)
import jax
import jax.numpy as jnp
from jax.experimental import pallas as pl


def kernel(rate_src, rate_dst, social_src, social_dst, pos_src, pos_dst, neg_src, neg_dst, user_emb, item_emb, W1s, b1s, W2s, b2s, W1p, b1p, W2p, b2p):
    raise NotImplementedError("write your pallas kernel here")


# speedup vs baseline: 1.2725x; 1.2725x over previous
import functools

import jax
import jax.numpy as jnp
from jax import lax
from jax.experimental import pallas as pl
from jax.experimental.pallas import tpu as pltpu
from jax.experimental.pallas import tpu_sc as plsc

N_NODES = 50000
D = 128
NC = 2
NS = 16
LANES = 16
BLK = 128

CHUNK = 12800
OUT_ROWS = 4 * CHUNK
ACC_ROWS = CHUNK + LANES
TRASH = CHUNK

BC_STRIPE = 3128
BC_ROWS = NS * BC_STRIPE
BC_TRASH = N_NODES


def _pad_edges(idx, mult, fill):
    e = idx.shape[0]
    ep = ((e + mult - 1) // mult) * mult
    return jnp.pad(idx.astype(jnp.int32), (0, ep - e), constant_values=fill)


def _mesh():
    return plsc.VectorSubcoreMesh(core_axis_name="c", subcore_axis_name="s",
                                  num_cores=NC, num_subcores=NS)


def _bincount_sc(idx_p):
    epad = idx_p.shape[0]
    per_tile = epad // (NC * NS)
    nblk = per_tile // BLK

    @functools.partial(
        pl.kernel,
        out_type=jax.ShapeDtypeStruct((NC, BC_ROWS, LANES), jnp.float32),
        mesh=_mesh(),
        compiler_params=pltpu.CompilerParams(use_tc_tiling_on_sc=False,
                                             needs_layout_passes=False),
        scratch_types=[
            pltpu.VMEM_SHARED((BC_ROWS, LANES), jnp.float32),
            pltpu.VMEM((BLK, LANES), jnp.float32),
            pltpu.VMEM((BC_STRIPE // 2, LANES), jnp.float32),
            pltpu.VMEM((BLK,), jnp.int32),
            pltpu.VMEM((BLK,), jnp.int32),
        ],
    )
    def k(idx_hbm, out_hbm, acc, ones_v, zed, dv, dloc):
        c = lax.axis_index("c")
        s = lax.axis_index("s")
        lane = lax.iota(jnp.int32, LANES)
        onerow = jnp.where(lane == 0, 1.0, 0.0).astype(jnp.float32)
        zrow = jnp.zeros((LANES,), jnp.float32)

        def fill(i, _):
            ones_v[i] = onerow
            return 0
        lax.fori_loop(0, BLK, fill, 0)

        def fillz(i, _):
            zed[i] = zrow
            return 0
        lax.fori_loop(0, BC_STRIPE // 2, fillz, 0)

        for j in range(2):
            pltpu.sync_copy(
                zed, acc.at[pl.ds(s * BC_STRIPE + j * (BC_STRIPE // 2),
                                  BC_STRIPE // 2)])
        plsc.subcore_barrier()

        wid = s * NC + c
        base0 = wid * per_tile

        def body(b, _):
            base = base0 + b * BLK
            pltpu.sync_copy(idx_hbm.at[pl.ds(base, BLK)], dv)
            for i in range(BLK // LANES):
                d = dv[pl.ds(i * LANES, LANES)]
                ok = (d >= 0) & (d < N_NODES)
                dloc[pl.ds(i * LANES, LANES)] = jnp.where(ok, d, BC_TRASH)
            pltpu.sync_copy(ones_v, acc.at[dloc], add=True)
            return 0
        lax.fori_loop(0, nblk, body, 0)
        plsc.subcore_barrier()
        pltpu.sync_copy(acc.at[pl.ds(s * BC_STRIPE, BC_STRIPE)],
                        out_hbm.at[c, pl.ds(s * BC_STRIPE, BC_STRIPE)])

    return k(idx_p)


ZR = 89


def _conv_sc(emb, src_p, dst_p):
    epad = src_p.shape[0]
    per_tile = epad // NS
    nblk = per_tile // BLK

    @functools.partial(
        pl.kernel,
        out_type=jax.ShapeDtypeStruct((OUT_ROWS, D), jnp.float32),
        mesh=_mesh(),
        compiler_params=pltpu.CompilerParams(use_tc_tiling_on_sc=False,
                                             needs_layout_passes=False),
        scratch_types=[
            pltpu.VMEM_SHARED((ACC_ROWS, D), jnp.float32),
            pltpu.VMEM((ZR, D), jnp.float32),
            pltpu.VMEM((BLK, D), jnp.float32),
            pltpu.VMEM((BLK,), jnp.int32),
            pltpu.VMEM((BLK,), jnp.int32),
            pltpu.VMEM((BLK,), jnp.int32),
            pltpu.SemaphoreType.DMA,
        ],
    )
    def k(emb_hbm, src_hbm, dst_hbm, out_hbm, acc, zed, rows, sv, dv, dloc,
          sem):
        c = lax.axis_index("c")
        s = lax.axis_index("s")
        zrow = jnp.zeros((LANES,), jnp.float32)

        def fillz(i, _):
            for kk in range(D // LANES):
                zed[i, pl.ds(kk * LANES, LANES)] = zrow
            return 0
        lax.fori_loop(0, ZR, fillz, 0)

        stripe = ACC_ROWS // NS
        for ci in range(2):
            chunk = 2 * c + ci
            lo = chunk * CHUNK
            hi = jnp.minimum(lo + CHUNK, N_NODES)
            for j in range(stripe // ZR):
                pltpu.sync_copy(
                    zed, acc.at[pl.ds(s * stripe + j * ZR, ZR)])
            plsc.subcore_barrier()

            base0 = s * per_tile

            def body(b, _):
                base = base0 + b * BLK
                pltpu.sync_copy(src_hbm.at[pl.ds(base, BLK)], sv)
                pltpu.sync_copy(dst_hbm.at[pl.ds(base, BLK)], dv)
                for i in range(BLK // LANES):
                    d = dv[pl.ds(i * LANES, LANES)]
                    ok = (d >= lo) & (d < hi)
                    dloc[pl.ds(i * LANES, LANES)] = jnp.where(ok, d - lo,
                                                              TRASH)
                pltpu.async_copy(emb_hbm.at[sv], rows, sem).wait()
                pltpu.sync_copy(rows, acc.at[dloc], add=True)
                return 0
            lax.fori_loop(0, nblk, body, 0)
            plsc.subcore_barrier()
            pltpu.sync_copy(
                acc.at[pl.ds(s * (CHUNK // NS), CHUNK // NS)],
                out_hbm.at[pl.ds(chunk * CHUNK + s * (CHUNK // NS),
                                 CHUNK // NS)])
            plsc.subcore_barrier()

    return k(emb, src_p, dst_p)


def _dot_sc(a, b, ai_p, bi_p, scale):
    epad = ai_p.shape[0]
    per_tile = epad // (NC * NS)
    nblk = per_tile // BLK

    @functools.partial(
        pl.kernel,
        out_type=jax.ShapeDtypeStruct((epad,), jnp.float32),
        mesh=_mesh(),
        compiler_params=pltpu.CompilerParams(use_tc_tiling_on_sc=False,
                                             needs_layout_passes=False),
        scratch_types=[
            pltpu.VMEM((BLK,), jnp.int32),
            pltpu.VMEM((BLK,), jnp.int32),
            pltpu.VMEM((BLK, D), jnp.float32),
            pltpu.VMEM((BLK, D), jnp.float32),
            pltpu.VMEM((BLK,), jnp.float32),
            pltpu.SemaphoreType.DMA,
            pltpu.SemaphoreType.DMA,
        ],
    )
    def k(a_hbm, b_hbm, ai_hbm, bi_hbm, out_hbm, av, bv, ra, rb, ov, sa, sb):
        c = lax.axis_index("c")
        s = lax.axis_index("s")
        wid = s * NC + c
        base0 = wid * per_tile
        lane = lax.iota(jnp.int32, LANES)

        def body(bk, _):
            base = base0 + bk * BLK
            pltpu.sync_copy(ai_hbm.at[pl.ds(base, BLK)], av)
            pltpu.sync_copy(bi_hbm.at[pl.ds(base, BLK)], bv)
            cp_a = pltpu.async_copy(a_hbm.at[av], ra, sa)
            cp_b = pltpu.async_copy(b_hbm.at[bv], rb, sb)
            cp_a.wait()
            cp_b.wait()
            for g in range(BLK // LANES):
                acc16 = jnp.zeros((LANES,), jnp.float32)
                for e in range(LANES):
                    edge = g * LANES + e
                    p = ra[edge, pl.ds(0, LANES)] * rb[edge, pl.ds(0, LANES)]
                    for kk in range(1, D // LANES):
                        p = p + (ra[edge, pl.ds(kk * LANES, LANES)]
                                 * rb[edge, pl.ds(kk * LANES, LANES)])
                    sval = jnp.sum(p) * scale
                    acc16 = jnp.where(lane == e, sval, acc16)
                ov[pl.ds(g * LANES, LANES)] = acc16
            pltpu.sync_copy(ov, out_hbm.at[pl.ds(base, BLK)])
            return 0
        lax.fori_loop(0, nblk, body, 0)

    return k(a, b, ai_p, bi_p)


RB = 512
GRID = OUT_ROWS // RB


def _prep_tc(user_p, item_p, rsru, rssu, rsri):
    def body(u_ref, i_ref, a_ref, b_ref, c_ref, o1, o2, o3):
        u = u_ref[...]
        o1[...] = u * a_ref[...]
        o2[...] = u * b_ref[...]
        o3[...] = i_ref[...] * c_ref[...]

    sds = jax.ShapeDtypeStruct((OUT_ROWS, D), jnp.float32)
    bs_e = pl.BlockSpec((RB, D), lambda i: (i, 0))
    bs_s = pl.BlockSpec((RB, 1), lambda i: (i, 0))
    return pl.pallas_call(
        body,
        grid=(GRID,),
        in_specs=[bs_e, bs_e, bs_s, bs_s, bs_s],
        out_specs=[bs_e, bs_e, bs_e],
        out_shape=[sds, sds, sds],
    )(user_p, item_p, rsru, rssu, rsri)


def _layer_user_tc(eu_raw, su_raw, res_u, rsru, rssu, rssd,
                   w1s, b1s, w2s, b2s, w1p, b1p, w2p, b2p):
    def body(eu_ref, su_ref, r_ref, a_ref, b_ref, d_ref,
             w1s_r, b1s_r, w2s_r, b2s_r, w1p_r, b1p_r, w2p_r, b2p_r,
             o_res, o_rate, o_soc):
        su = su_ref[...] * d_ref[...]
        eup = eu_ref[...] * a_ref[...]
        hs = jnp.tanh(jnp.dot(su, w1s_r[...],
                              preferred_element_type=jnp.float32) + b1s_r[...])
        fs = jnp.dot(hs, w2s_r[...],
                     preferred_element_type=jnp.float32) + b2s_r[...]
        hp = jnp.tanh(jnp.dot(eup, w1p_r[...],
                              preferred_element_type=jnp.float32) + b1p_r[...])
        fp = jnp.dot(hp, w2p_r[...],
                     preferred_element_type=jnp.float32) + b2p_r[...]
        eu = fs + fp
        nrm = jnp.sqrt(jnp.sum(eu * eu, axis=1, keepdims=True))
        eu = eu / jnp.maximum(nrm, 1e-12)
        o_res[...] = r_ref[...] + eu
        o_rate[...] = eu * a_ref[...]
        o_soc[...] = eu * b_ref[...]

    sds = jax.ShapeDtypeStruct((OUT_ROWS, D), jnp.float32)
    bs_e = pl.BlockSpec((RB, D), lambda i: (i, 0))
    bs_s = pl.BlockSpec((RB, 1), lambda i: (i, 0))
    bs_w = pl.BlockSpec((D, D), lambda i: (0, 0))
    bs_b = pl.BlockSpec((1, D), lambda i: (0, 0))
    return pl.pallas_call(
        body,
        grid=(GRID,),
        in_specs=[bs_e, bs_e, bs_e, bs_s, bs_s, bs_s,
                  bs_w, bs_b, bs_w, bs_b, bs_w, bs_b, bs_w, bs_b],
        out_specs=[bs_e, bs_e, bs_e],
        out_shape=[sds, sds, sds],
    )(eu_raw, su_raw, res_u, rsru, rssu, rssd,
      w1s, b1s, w2s, b2s, w1p, b1p, w2p, b2p)


def _layer_item_tc(ei_raw, res_i, rsri):
    def body(e_ref, r_ref, c_ref, o_res, o_next):
        ei = e_ref[...] * c_ref[...]
        o_res[...] = r_ref[...] + ei
        o_next[...] = ei * c_ref[...]

    sds = jax.ShapeDtypeStruct((OUT_ROWS, D), jnp.float32)
    bs_e = pl.BlockSpec((RB, D), lambda i: (i, 0))
    bs_s = pl.BlockSpec((RB, 1), lambda i: (i, 0))
    return pl.pallas_call(
        body,
        grid=(GRID,),
        in_specs=[bs_e, bs_e, bs_s],
        out_specs=[bs_e, bs_e],
        out_shape=[sds, sds],
    )(ei_raw, res_i, rsri)


def kernel(rate_src, rate_dst, social_src, social_dst, pos_src, pos_dst,
           neg_src, neg_dst, user_emb, item_emb,
           W1s, b1s, W2s, b2s, W1p, b1p, W2p, b2p):
    emult = NC * NS * BLK
    r_src0 = _pad_edges(rate_src, emult, 0)
    r_srcm = _pad_edges(rate_src, emult, -1)
    r_dst0 = _pad_edges(rate_dst, emult, 0)
    r_dstm = _pad_edges(rate_dst, emult, -1)
    s_src0 = _pad_edges(social_src, emult, 0)
    s_srcm = _pad_edges(social_src, emult, -1)
    s_dstm = _pad_edges(social_dst, emult, -1)
    p_src = _pad_edges(pos_src, emult, 0)
    p_dst = _pad_edges(pos_dst, emult, 0)
    n_src = _pad_edges(neg_src, emult, 0)
    n_dst = _pad_edges(neg_dst, emult, 0)

    def rs_of(partials):
        deg = partials[0, :N_NODES, 0] + partials[1, :N_NODES, 0]
        rs = jax.lax.rsqrt(jnp.maximum(deg, 1.0))
        return jnp.pad(rs, (0, OUT_ROWS - N_NODES))[:, None]

    rs_ru = rs_of(_bincount_sc(r_srcm))
    rs_ri = rs_of(_bincount_sc(r_dstm))
    rs_su = rs_of(_bincount_sc(s_srcm))
    rs_sd = rs_of(_bincount_sc(s_dstm))

    pad_n = ((0, OUT_ROWS - N_NODES), (0, 0))
    user_p = jnp.pad(user_emb, pad_n)
    item_p = jnp.pad(item_emb, pad_n)

    u_rate, u_soc, i_rate = _prep_tc(user_p, item_p, rs_ru, rs_su, rs_ri)
    res_u, res_i = user_p, item_p
    w1sT, w2sT, w1pT, w2pT = W1s.T, W2s.T, W1p.T, W2p.T
    b1s2, b2s2 = b1s[None, :], b2s[None, :]
    b1p2, b2p2 = b1p[None, :], b2p[None, :]

    for layer in range(2):
        eu_raw = _conv_sc(i_rate, r_dst0, r_srcm)
        ei_raw = _conv_sc(u_rate, r_src0, r_dstm)
        su_raw = _conv_sc(u_soc, s_src0, s_dstm)
        res_u, u_rate, u_soc = _layer_user_tc(
            eu_raw, su_raw, res_u, rs_ru, rs_su, rs_sd,
            w1sT, b1s2, w2sT, b2s2, w1pT, b1p2, w2pT, b2p2)
        res_i, i_rate = _layer_item_tc(ei_raw, res_i, rs_ri)

    scale = 1.0 / 9.0
    pos = _dot_sc(res_u, res_i, p_src, p_dst, scale)[:pos_src.shape[0], None]
    neg = _dot_sc(res_u, res_i, n_src, n_dst, scale)[:neg_src.shape[0], None]
    return (pos, neg)


# speedup vs baseline: 1.3998x; 1.1000x over previous
import functools

import jax
import jax.numpy as jnp
from jax import lax
from jax.experimental import pallas as pl
from jax.experimental.pallas import tpu as pltpu
from jax.experimental.pallas import tpu_sc as plsc

N_NODES = 50000
D = 128
NC = 2
NS = 16
LANES = 16
BLK = 128

CHUNK = 12800
OUT_ROWS = 4 * CHUNK
ACC_ROWS = CHUNK + LANES
TRASH = CHUNK

BC_STRIPE = 3128
BC_ROWS = NS * BC_STRIPE
BC_TRASH = N_NODES


def _pad_edges(idx, mult, fill):
    e = idx.shape[0]
    ep = ((e + mult - 1) // mult) * mult
    return jnp.pad(idx.astype(jnp.int32), (0, ep - e), constant_values=fill)


def _mesh():
    return plsc.VectorSubcoreMesh(core_axis_name="c", subcore_axis_name="s",
                                  num_cores=NC, num_subcores=NS)


def _bincount_sc(idx_p):
    epad = idx_p.shape[0]
    per_tile = epad // (NC * NS)
    nblk = per_tile // BLK

    @functools.partial(
        pl.kernel,
        out_type=jax.ShapeDtypeStruct((NC, BC_ROWS, LANES), jnp.float32),
        mesh=_mesh(),
        compiler_params=pltpu.CompilerParams(use_tc_tiling_on_sc=False,
                                             needs_layout_passes=False),
        scratch_types=[
            pltpu.VMEM_SHARED((BC_ROWS, LANES), jnp.float32),
            pltpu.VMEM((BLK, LANES), jnp.float32),
            pltpu.VMEM((BC_STRIPE // 2, LANES), jnp.float32),
            pltpu.VMEM((BLK,), jnp.int32),
            pltpu.VMEM((BLK,), jnp.int32),
        ],
    )
    def k(idx_hbm, out_hbm, acc, ones_v, zed, dv, dloc):
        c = lax.axis_index("c")
        s = lax.axis_index("s")
        lane = lax.iota(jnp.int32, LANES)
        onerow = jnp.where(lane == 0, 1.0, 0.0).astype(jnp.float32)
        zrow = jnp.zeros((LANES,), jnp.float32)

        def fill(i, _):
            ones_v[i] = onerow
            return 0
        lax.fori_loop(0, BLK, fill, 0)

        def fillz(i, _):
            zed[i] = zrow
            return 0
        lax.fori_loop(0, BC_STRIPE // 2, fillz, 0)

        for j in range(2):
            pltpu.sync_copy(
                zed, acc.at[pl.ds(s * BC_STRIPE + j * (BC_STRIPE // 2),
                                  BC_STRIPE // 2)])
        plsc.subcore_barrier()

        wid = s * NC + c
        base0 = wid * per_tile

        def body(b, _):
            base = base0 + b * BLK
            pltpu.sync_copy(idx_hbm.at[pl.ds(base, BLK)], dv)
            for i in range(BLK // LANES):
                d = dv[pl.ds(i * LANES, LANES)]
                ok = (d >= 0) & (d < N_NODES)
                dloc[pl.ds(i * LANES, LANES)] = jnp.where(ok, d, BC_TRASH)
            pltpu.sync_copy(ones_v, acc.at[dloc], add=True)
            return 0
        lax.fori_loop(0, nblk, body, 0)
        plsc.subcore_barrier()
        pltpu.sync_copy(acc.at[pl.ds(s * BC_STRIPE, BC_STRIPE)],
                        out_hbm.at[c, pl.ds(s * BC_STRIPE, BC_STRIPE)])

    return k(idx_p)


SBLK = 2048
GBLK = 64


def _conv_sc(emb, src_p, dst_p):
    epad = src_p.shape[0]
    per_tile = epad // NS
    nsb = per_tile // SBLK
    ncap = SBLK + GBLK
    stripe = ACC_ROWS // NS

    @functools.partial(
        pl.kernel,
        out_type=jax.ShapeDtypeStruct((OUT_ROWS, D), jnp.float32),
        mesh=_mesh(),
        compiler_params=pltpu.CompilerParams(use_tc_tiling_on_sc=False,
                                             needs_layout_passes=False),
        scratch_types=[
            pltpu.VMEM_SHARED((ACC_ROWS, D), jnp.float32),
            pltpu.VMEM((ncap,), jnp.int32),
            pltpu.VMEM((ncap,), jnp.int32),
            pltpu.VMEM((GBLK, D), jnp.float32),
            pltpu.VMEM((GBLK, D), jnp.float32),
            pltpu.VMEM((SBLK,), jnp.int32),
            pltpu.VMEM((SBLK,), jnp.int32),
            pltpu.VMEM((GBLK,), jnp.int32),
            pltpu.VMEM((GBLK,), jnp.int32),
            pltpu.VMEM((GBLK,), jnp.int32),
            pltpu.VMEM((GBLK,), jnp.int32),
            pltpu.SemaphoreType.DMA,
            pltpu.SemaphoreType.DMA,
        ],
    )
    def k(emb_hbm, src_hbm, dst_hbm, out_hbm, acc, buf_s, buf_d,
          rows0, rows1, svb, dvb, st_s0, st_d0, st_s1, st_d1, sem0, sem1):
        c = lax.axis_index("c")
        s = lax.axis_index("s")
        zrow = jnp.zeros((LANES,), jnp.float32)
        lane = lax.iota(jnp.int32, LANES)

        for ci in range(2):
            chunk = 2 * c + ci
            lo = chunk * CHUNK
            hi = jnp.minimum(lo + CHUNK, N_NODES)

            def zr(r, _):
                for kk in range(D // LANES):
                    rows0[r, pl.ds(kk * LANES, LANES)] = zrow
                return 0
            lax.fori_loop(0, GBLK, zr, 0)
            for j in range(stripe // GBLK):
                pltpu.sync_copy(rows0,
                                acc.at[pl.ds(s * stripe + j * GBLK, GBLK)])
            pltpu.sync_copy(
                rows0.at[pl.ds(0, stripe % GBLK)],
                acc.at[pl.ds(s * stripe + (stripe // GBLK) * GBLK,
                             stripe % GBLK)])
            plsc.subcore_barrier()

            def stage(g, sts, std):
                off = pl.multiple_of(g * GBLK, GBLK)
                for kk in range(GBLK // LANES):
                    sts[pl.ds(kk * LANES, LANES)] = (
                        buf_s[pl.ds(off + kk * LANES, LANES)])
                    std[pl.ds(kk * LANES, LANES)] = (
                        buf_d[pl.ds(off + kk * LANES, LANES)])

            def scan_block(b, _):
                base = s * per_tile + b * SBLK
                pltpu.sync_copy(src_hbm.at[pl.ds(base, SBLK)], svb)
                pltpu.sync_copy(dst_hbm.at[pl.ds(base, SBLK)], dvb)

                def group(g, kc):
                    d = dvb[pl.ds(g * LANES, LANES)]
                    sv = svb[pl.ds(g * LANES, LANES)]
                    ok = (d >= lo) & (d < hi)
                    oki = jnp.where(ok, 1, 0).astype(jnp.int32)
                    incl = plsc.cumsum(oki)
                    pos = kc + incl - oki
                    plsc.store_scatter(buf_s, [pos], sv, mask=ok)
                    plsc.store_scatter(buf_d, [pos], d - lo, mask=ok)
                    return kc + jnp.sum(oki)
                kc = lax.fori_loop(0, SBLK // LANES, group, jnp.int32(0))

                for j in range(GBLK // LANES):
                    pos = kc + lane + j * LANES
                    plsc.store_scatter(buf_s, [pos],
                                       jnp.zeros((LANES,), jnp.int32))
                    plsc.store_scatter(buf_d, [pos],
                                       jnp.full((LANES,), TRASH, jnp.int32))
                nb = lax.shift_right_logical(kc + (GBLK - 1), 6)

                @pl.when(nb > 0)
                def _():
                    stage(0, st_s0, st_d0)
                    pltpu.async_copy(emb_hbm.at[st_s0], rows0, sem0)

                def pair(p, _):
                    g0 = 2 * p
                    g1 = g0 + 1

                    @pl.when(g1 < nb)
                    def _():
                        stage(g1, st_s1, st_d1)
                        pltpu.async_copy(emb_hbm.at[st_s1], rows1, sem1)
                    pltpu.make_async_copy(emb_hbm.at[st_s0], rows0,
                                          sem0).wait()
                    pltpu.sync_copy(rows0, acc.at[st_d0], add=True)

                    @pl.when(g1 + 1 < nb)
                    def _():
                        stage(g1 + 1, st_s0, st_d0)
                        pltpu.async_copy(emb_hbm.at[st_s0], rows0, sem0)

                    @pl.when(g1 < nb)
                    def _():
                        pltpu.make_async_copy(emb_hbm.at[st_s1], rows1,
                                              sem1).wait()
                        pltpu.sync_copy(rows1, acc.at[st_d1], add=True)
                    return 0
                lax.fori_loop(0, lax.shift_right_logical(nb + 1, 1), pair, 0)
                return 0
            lax.fori_loop(0, nsb, scan_block, 0)
            plsc.subcore_barrier()
            pltpu.sync_copy(
                acc.at[pl.ds(s * (CHUNK // NS), CHUNK // NS)],
                out_hbm.at[pl.ds(chunk * CHUNK + s * (CHUNK // NS),
                                 CHUNK // NS)])
            plsc.subcore_barrier()

    return k(emb, src_p, dst_p)


def _dot_sc(a, b, ai_p, bi_p, scale):
    epad = ai_p.shape[0]
    per_tile = epad // (NC * NS)
    nblk = per_tile // BLK

    @functools.partial(
        pl.kernel,
        out_type=jax.ShapeDtypeStruct((epad,), jnp.float32),
        mesh=_mesh(),
        compiler_params=pltpu.CompilerParams(use_tc_tiling_on_sc=False,
                                             needs_layout_passes=False),
        scratch_types=[
            pltpu.VMEM((BLK,), jnp.int32),
            pltpu.VMEM((BLK,), jnp.int32),
            pltpu.VMEM((BLK, D), jnp.float32),
            pltpu.VMEM((BLK, D), jnp.float32),
            pltpu.VMEM((BLK,), jnp.float32),
            pltpu.SemaphoreType.DMA,
            pltpu.SemaphoreType.DMA,
        ],
    )
    def k(a_hbm, b_hbm, ai_hbm, bi_hbm, out_hbm, av, bv, ra, rb, ov, sa, sb):
        c = lax.axis_index("c")
        s = lax.axis_index("s")
        wid = s * NC + c
        base0 = wid * per_tile
        lane = lax.iota(jnp.int32, LANES)

        def body(bk, _):
            base = base0 + bk * BLK
            pltpu.sync_copy(ai_hbm.at[pl.ds(base, BLK)], av)
            pltpu.sync_copy(bi_hbm.at[pl.ds(base, BLK)], bv)
            cp_a = pltpu.async_copy(a_hbm.at[av], ra, sa)
            cp_b = pltpu.async_copy(b_hbm.at[bv], rb, sb)
            cp_a.wait()
            cp_b.wait()
            for g in range(BLK // LANES):
                acc16 = jnp.zeros((LANES,), jnp.float32)
                for e in range(LANES):
                    edge = g * LANES + e
                    p = ra[edge, pl.ds(0, LANES)] * rb[edge, pl.ds(0, LANES)]
                    for kk in range(1, D // LANES):
                        p = p + (ra[edge, pl.ds(kk * LANES, LANES)]
                                 * rb[edge, pl.ds(kk * LANES, LANES)])
                    sval = jnp.sum(p) * scale
                    acc16 = jnp.where(lane == e, sval, acc16)
                ov[pl.ds(g * LANES, LANES)] = acc16
            pltpu.sync_copy(ov, out_hbm.at[pl.ds(base, BLK)])
            return 0
        lax.fori_loop(0, nblk, body, 0)

    return k(a, b, ai_p, bi_p)


RB = 512
GRID = OUT_ROWS // RB


def _prep_tc(user_p, item_p, rsru, rssu, rsri):
    def body(u_ref, i_ref, a_ref, b_ref, c_ref, o1, o2, o3):
        u = u_ref[...]
        o1[...] = u * a_ref[...]
        o2[...] = u * b_ref[...]
        o3[...] = i_ref[...] * c_ref[...]

    sds = jax.ShapeDtypeStruct((OUT_ROWS, D), jnp.float32)
    bs_e = pl.BlockSpec((RB, D), lambda i: (i, 0))
    bs_s = pl.BlockSpec((RB, 1), lambda i: (i, 0))
    return pl.pallas_call(
        body,
        grid=(GRID,),
        in_specs=[bs_e, bs_e, bs_s, bs_s, bs_s],
        out_specs=[bs_e, bs_e, bs_e],
        out_shape=[sds, sds, sds],
    )(user_p, item_p, rsru, rssu, rsri)


def _layer_user_tc(eu_raw, su_raw, res_u, rsru, rssu, rssd,
                   w1s, b1s, w2s, b2s, w1p, b1p, w2p, b2p):
    def body(eu_ref, su_ref, r_ref, a_ref, b_ref, d_ref,
             w1s_r, b1s_r, w2s_r, b2s_r, w1p_r, b1p_r, w2p_r, b2p_r,
             o_res, o_rate, o_soc):
        su = su_ref[...] * d_ref[...]
        eup = eu_ref[...] * a_ref[...]
        hs = jnp.tanh(jnp.dot(su, w1s_r[...],
                              preferred_element_type=jnp.float32) + b1s_r[...])
        fs = jnp.dot(hs, w2s_r[...],
                     preferred_element_type=jnp.float32) + b2s_r[...]
        hp = jnp.tanh(jnp.dot(eup, w1p_r[...],
                              preferred_element_type=jnp.float32) + b1p_r[...])
        fp = jnp.dot(hp, w2p_r[...],
                     preferred_element_type=jnp.float32) + b2p_r[...]
        eu = fs + fp
        nrm = jnp.sqrt(jnp.sum(eu * eu, axis=1, keepdims=True))
        eu = eu / jnp.maximum(nrm, 1e-12)
        o_res[...] = r_ref[...] + eu
        o_rate[...] = eu * a_ref[...]
        o_soc[...] = eu * b_ref[...]

    sds = jax.ShapeDtypeStruct((OUT_ROWS, D), jnp.float32)
    bs_e = pl.BlockSpec((RB, D), lambda i: (i, 0))
    bs_s = pl.BlockSpec((RB, 1), lambda i: (i, 0))
    bs_w = pl.BlockSpec((D, D), lambda i: (0, 0))
    bs_b = pl.BlockSpec((1, D), lambda i: (0, 0))
    return pl.pallas_call(
        body,
        grid=(GRID,),
        in_specs=[bs_e, bs_e, bs_e, bs_s, bs_s, bs_s,
                  bs_w, bs_b, bs_w, bs_b, bs_w, bs_b, bs_w, bs_b],
        out_specs=[bs_e, bs_e, bs_e],
        out_shape=[sds, sds, sds],
    )(eu_raw, su_raw, res_u, rsru, rssu, rssd,
      w1s, b1s, w2s, b2s, w1p, b1p, w2p, b2p)


def _layer_item_tc(ei_raw, res_i, rsri):
    def body(e_ref, r_ref, c_ref, o_res, o_next):
        ei = e_ref[...] * c_ref[...]
        o_res[...] = r_ref[...] + ei
        o_next[...] = ei * c_ref[...]

    sds = jax.ShapeDtypeStruct((OUT_ROWS, D), jnp.float32)
    bs_e = pl.BlockSpec((RB, D), lambda i: (i, 0))
    bs_s = pl.BlockSpec((RB, 1), lambda i: (i, 0))
    return pl.pallas_call(
        body,
        grid=(GRID,),
        in_specs=[bs_e, bs_e, bs_s],
        out_specs=[bs_e, bs_e],
        out_shape=[sds, sds],
    )(ei_raw, res_i, rsri)


def kernel(rate_src, rate_dst, social_src, social_dst, pos_src, pos_dst,
           neg_src, neg_dst, user_emb, item_emb,
           W1s, b1s, W2s, b2s, W1p, b1p, W2p, b2p):
    emult = NC * NS * BLK
    cmult = NS * SBLK
    r_src0 = _pad_edges(rate_src, cmult, 0)
    r_srcm = _pad_edges(rate_src, cmult, -1)
    r_dst0 = _pad_edges(rate_dst, cmult, 0)
    r_dstm = _pad_edges(rate_dst, cmult, -1)
    s_src0 = _pad_edges(social_src, cmult, 0)
    s_srcm = _pad_edges(social_src, cmult, -1)
    s_dstm = _pad_edges(social_dst, cmult, -1)
    p_src = _pad_edges(pos_src, emult, 0)
    p_dst = _pad_edges(pos_dst, emult, 0)
    n_src = _pad_edges(neg_src, emult, 0)
    n_dst = _pad_edges(neg_dst, emult, 0)

    def rs_of(partials):
        deg = partials[0, :N_NODES, 0] + partials[1, :N_NODES, 0]
        rs = jax.lax.rsqrt(jnp.maximum(deg, 1.0))
        return jnp.pad(rs, (0, OUT_ROWS - N_NODES))[:, None]

    rs_ru = rs_of(_bincount_sc(r_srcm))
    rs_ri = rs_of(_bincount_sc(r_dstm))
    rs_su = rs_of(_bincount_sc(s_srcm))
    rs_sd = rs_of(_bincount_sc(s_dstm))

    pad_n = ((0, OUT_ROWS - N_NODES), (0, 0))
    user_p = jnp.pad(user_emb, pad_n)
    item_p = jnp.pad(item_emb, pad_n)

    u_rate, u_soc, i_rate = _prep_tc(user_p, item_p, rs_ru, rs_su, rs_ri)
    res_u, res_i = user_p, item_p
    w1sT, w2sT, w1pT, w2pT = W1s.T, W2s.T, W1p.T, W2p.T
    b1s2, b2s2 = b1s[None, :], b2s[None, :]
    b1p2, b2p2 = b1p[None, :], b2p[None, :]

    for layer in range(2):
        eu_raw = _conv_sc(i_rate, r_dst0, r_srcm)
        ei_raw = _conv_sc(u_rate, r_src0, r_dstm)
        su_raw = _conv_sc(u_soc, s_src0, s_dstm)
        res_u, u_rate, u_soc = _layer_user_tc(
            eu_raw, su_raw, res_u, rs_ru, rs_su, rs_sd,
            w1sT, b1s2, w2sT, b2s2, w1pT, b1p2, w2pT, b2p2)
        res_i, i_rate = _layer_item_tc(ei_raw, res_i, rs_ri)

    scale = 1.0 / 9.0
    pos = _dot_sc(res_u, res_i, p_src, p_dst, scale)[:pos_src.shape[0], None]
    neg = _dot_sc(res_u, res_i, n_src, n_dst, scale)[:neg_src.shape[0], None]
    return (pos, neg)
